# Initial kernel scaffold; baseline (speedup 1.0000x reference)
import jax
import jax.numpy as jnp
from jax.experimental import pallas as pl


def kernel(x, edge_index, W1, b1, W2, b2):
    raise NotImplementedError("write your pallas kernel here")


# speedup vs baseline: 4.2622x; 4.2622x over previous
import functools

import jax
import jax.numpy as jnp
from jax import lax
from jax.experimental import pallas as pl
from jax.experimental.pallas import tpu as pltpu
from jax.experimental.pallas import tpu_sc as plsc

C = 1.0
EPS = 1e-6

NC = 2
NS = 16
NW = NC * NS
B_EDGE = 80


def _col_mask(shape):
    return lax.broadcasted_iota(jnp.int32, shape, 1) == 0


def _arccosh(z):
    return jnp.log(z + jnp.sqrt(z * z - 1.0))


def _logmap0_point(x, is0):
    y = jnp.where(is0, 0.0, x)
    x0 = jnp.sum(jnp.where(is0, x, 0.0), axis=1, keepdims=True)
    yn = jnp.sqrt(jnp.sum(y * y, axis=1, keepdims=True) + EPS)
    th = _arccosh(jnp.maximum(x0, 1.0 + EPS))
    return th * y / yn


def _exp_proj_log(t):
    tn = jnp.sqrt(jnp.sum(t * t, axis=1, keepdims=True) + EPS)
    e = jnp.exp(tn)
    einv = 1.0 / e
    sinh = 0.5 * (e - einv)
    resr = sinh * t / tn
    rs = jnp.sum(resr * resr, axis=1, keepdims=True)
    x0 = jnp.sqrt(1.0 + rs)
    yn = jnp.sqrt(rs + EPS)
    th = _arccosh(jnp.maximum(x0, 1.0 + EPS))
    return th * resr / yn


def _dense1_body(x_ref, w_ref, b_ref, o_ref):
    x = x_ref[...]
    is0 = _col_mask(x.shape)
    u = _logmap0_point(x, is0)
    mu = jnp.dot(u, w_ref[...], preferred_element_type=jnp.float32) + b_ref[...]
    mu = jnp.where(is0, 0.0, mu)
    o_ref[...] = _exp_proj_log(mu)


def _deg_scale(d0, d1):
    isc0 = _col_mask(d0.shape)
    deg = jnp.sum(jnp.where(isc0, d0 + d1, 0.0), axis=1, keepdims=True)
    return 1.0 / jnp.maximum(deg, 1.0)


def _dense2_body(p_ref, d0_ref, d1_ref, w_ref, b_ref, o_ref):
    agg = p_ref[...]
    scale = _deg_scale(d0_ref[...], d1_ref[...])
    is0 = _col_mask(agg.shape)
    t = jnp.where(is0, 0.0, agg * scale)
    t = _exp_proj_log(t)
    t = jnp.where(is0, 0.0, jnp.maximum(t, 0.0))
    t = _exp_proj_log(t)
    mu = jnp.dot(t, w_ref[...], preferred_element_type=jnp.float32) + b_ref[...]
    mu = jnp.where(is0, 0.0, mu)
    o_ref[...] = _exp_proj_log(mu)


def _dense3_body(p_ref, d0_ref, d1_ref, o_ref):
    agg = p_ref[...]
    scale = _deg_scale(d0_ref[...], d1_ref[...])
    is0 = _col_mask(agg.shape)
    t = jnp.where(is0, 0.0, agg * scale)
    t = _exp_proj_log(t)
    m = jnp.max(t, axis=1, keepdims=True)
    lse = jnp.log(jnp.sum(jnp.exp(t - m), axis=1, keepdims=True))
    o_ref[...] = t - m - lse


def _row_spec(rb):
    return pl.BlockSpec((rb, 128), lambda i: (i, 0))


def _fixed_spec(shape):
    return pl.BlockSpec(shape, lambda i: (0, 0))


def _dense1(x, w1t, b1, rb):
    n = x.shape[0]
    return pl.pallas_call(
        _dense1_body,
        grid=(n // rb,),
        in_specs=[_row_spec(rb), _fixed_spec((128, 128)), _fixed_spec((1, 128))],
        out_specs=_row_spec(rb),
        out_shape=jax.ShapeDtypeStruct((n, 128), jnp.float32),
    )(x, w1t, b1)


def _dense2(p, d0, d1, w2t, b2, rb, n):
    dspec = pl.BlockSpec((rb, 16), lambda i: (i, 0))
    return pl.pallas_call(
        _dense2_body,
        grid=(n // rb,),
        in_specs=[_row_spec(rb), dspec, dspec,
                  _fixed_spec((128, 128)), _fixed_spec((1, 128))],
        out_specs=_row_spec(rb),
        out_shape=jax.ShapeDtypeStruct((n, 128), jnp.float32),
    )(p, d0, d1, w2t, b2)


def _dense3(p, d0, d1, rb, n):
    dspec = pl.BlockSpec((rb, 16), lambda i: (i, 0))
    return pl.pallas_call(
        _dense3_body,
        grid=(n // rb,),
        in_specs=[_row_spec(rb), dspec, dspec],
        out_specs=_row_spec(rb),
        out_shape=jax.ShapeDtypeStruct((n, 128), jnp.float32),
    )(p, d0, d1)


def _npad(n):
    return ((n + NS * 8 - 1) // (NS * 8)) * NS * 8


@functools.partial(jax.jit, static_argnames=("n", "d", "e"))
def _segsum(src3, dst3, ht, *, n, d, e):
    nch = e // NS // B_EDGE
    half = _npad(n) // 2
    accr = ((half + 1 + NS * 8 - 1) // (NS * 8)) * NS * 8
    rpt = accr // NS
    mesh = plsc.VectorSubcoreMesh(core_axis_name="c", subcore_axis_name="s")

    epw = nch * B_EDGE
    out_type = [jax.ShapeDtypeStruct((NC, accr, d), jnp.float32)]
    scratch = [
        pltpu.VMEM((epw,), jnp.int32),
        pltpu.VMEM((epw,), jnp.int32),
        pltpu.VMEM((B_EDGE,), jnp.int32),
        pltpu.VMEM((B_EDGE, d), jnp.float32),
        pltpu.VMEM((64, d), jnp.float32),
        pltpu.VMEM_SHARED((accr, d), jnp.float32),
        pltpu.SemaphoreType.DMA,
    ]

    def body(src_hbm, dst_hbm, ht_hbm, out_hbm, sidx, didx, gidx, rows, zrow,
             acc, sem):
        c = lax.axis_index("c")
        s = lax.axis_index("s")
        lo = c * half

        zero16 = jnp.zeros((16,), jnp.float32)

        def zfill(i, carry):
            for j in range(d // 16):
                zrow[i, pl.ds(j * 16, 16)] = zero16
            return carry
        lax.fori_loop(0, 64, zfill, 0)

        base = s * rpt
        nfull, tail = rpt // 64, rpt % 64
        for k in range(nfull):
            pltpu.sync_copy(zrow, acc.at[pl.ds(base + k * 64, 64)])
        if tail:
            pltpu.sync_copy(zrow.at[pl.ds(0, tail)],
                            acc.at[pl.ds(base + nfull * 64, tail)])
        plsc.subcore_barrier()

        pltpu.sync_copy(src_hbm.at[s], sidx)
        pltpu.sync_copy(dst_hbm.at[s], didx)

        def eloop(i, carry):
            eb = i * B_EDGE
            for j in range(B_EDGE // 16):
                v = didx[pl.ds(eb + j * 16, 16)] - lo
                ok = (v >= 0) & (v < half)
                gidx[pl.ds(j * 16, 16)] = jnp.where(ok, v, half)
            pltpu.async_copy(ht_hbm.at[sidx.at[pl.ds(eb, B_EDGE)]], rows,
                             sem).wait()
            pltpu.sync_copy(rows, acc.at[gidx], add=True)
            return carry
        lax.fori_loop(0, nch, eloop, 0)
        plsc.subcore_barrier()

        pltpu.sync_copy(acc.at[pl.ds(base, rpt)],
                        out_hbm.at[c, pl.ds(base, rpt)])

    run = pl.kernel(body, out_type=out_type, mesh=mesh, scratch_types=scratch)
    return run(src3, dst3, ht)


@functools.partial(jax.jit, static_argnames=("n", "e"))
def _degree(dst3, *, n, e):
    nch = e // NW // B_EDGE
    npad = _npad(n)
    rpt = npad // NS
    zr = 128
    mesh = plsc.VectorSubcoreMesh(core_axis_name="c", subcore_axis_name="s")

    out_type = [jax.ShapeDtypeStruct((NC, npad, 16), jnp.float32)]
    scratch = [
        pltpu.VMEM((nch, B_EDGE), jnp.int32),
        pltpu.VMEM((B_EDGE, 16), jnp.float32),
        pltpu.VMEM((zr, 16), jnp.float32),
        pltpu.VMEM_SHARED((npad, 16), jnp.float32),
    ]

    def body(dst_hbm, deg_hbm, didx, ones, zdeg, dacc):
        c = lax.axis_index("c")
        s = lax.axis_index("s")
        wid = s * NC + c

        zero16 = jnp.zeros((16,), jnp.float32)
        one16 = jnp.ones((16,), jnp.float32)

        def zfill(i, carry):
            zdeg[i, pl.ds(0, 16)] = zero16
            return carry
        lax.fori_loop(0, zr, zfill, 0)

        def ofill(i, carry):
            ones[i, pl.ds(0, 16)] = one16
            return carry
        lax.fori_loop(0, B_EDGE, ofill, 0)

        base = s * rpt
        for k in range(rpt // zr):
            pltpu.sync_copy(zdeg, dacc.at[pl.ds(base + k * zr, zr)])
        plsc.subcore_barrier()

        pltpu.sync_copy(dst_hbm.at[wid], didx)

        def eloop(i, carry):
            pltpu.sync_copy(ones, dacc.at[didx.at[i]], add=True)
            return carry
        lax.fori_loop(0, nch, eloop, 0)
        plsc.subcore_barrier()

        pltpu.sync_copy(dacc.at[pl.ds(base, rpt)],
                        deg_hbm.at[c, pl.ds(base, rpt)])

    run = pl.kernel(body, out_type=out_type, mesh=mesh, scratch_types=scratch)
    return run(dst3)


def kernel(x, edge_index, W1, b1, W2, b2):
    n, d = x.shape
    e = edge_index.shape[1]
    rb = 1000

    src3 = edge_index[0].reshape(NS, e // NS)
    dst3 = edge_index[1].reshape(NS, e // NS)
    dst3d = edge_index[1].reshape(NW, e // NW // B_EDGE, B_EDGE)

    half = _npad(n) // 2

    ht1 = _dense1(x, W1.T, b1.reshape(1, -1), rb)
    (degp,) = _degree(dst3d, n=n, e=e)
    (p1,) = _segsum(src3, dst3, ht1, n=n, d=d, e=e)
    agg1 = jnp.concatenate([p1[0, :half], p1[1, :half]], axis=0)
    ht2 = _dense2(agg1, degp[0], degp[1], W2.T, b2.reshape(1, -1), rb, n)
    (p2,) = _segsum(src3, dst3, ht2, n=n, d=d, e=e)
    agg2 = jnp.concatenate([p2[0, :half], p2[1, :half]], axis=0)
    return _dense3(agg2, degp[0], degp[1], rb, n)


# speedup vs baseline: 10.5492x; 2.4751x over previous
import functools

import jax
import jax.numpy as jnp
from jax import lax
from jax.experimental import pallas as pl
from jax.experimental.pallas import tpu as pltpu
from jax.experimental.pallas import tpu_sc as plsc

C = 1.0
EPS = 1e-6

NC = 2
NS = 16
NW = NC * NS
B_EDGE = 80


def _col_mask(shape):
    return lax.broadcasted_iota(jnp.int32, shape, 1) == 0


def _arccosh(z):
    return jnp.log(z + jnp.sqrt(z * z - 1.0))


def _logmap0_point(x, is0):
    y = jnp.where(is0, 0.0, x)
    x0 = jnp.sum(jnp.where(is0, x, 0.0), axis=1, keepdims=True)
    yn = jnp.sqrt(jnp.sum(y * y, axis=1, keepdims=True) + EPS)
    th = _arccosh(jnp.maximum(x0, 1.0 + EPS))
    return th * y / yn


def _exp_proj_log(t):
    tn = jnp.sqrt(jnp.sum(t * t, axis=1, keepdims=True) + EPS)
    e = jnp.exp(tn)
    einv = 1.0 / e
    sinh = 0.5 * (e - einv)
    resr = sinh * t / tn
    rs = jnp.sum(resr * resr, axis=1, keepdims=True)
    x0 = jnp.sqrt(1.0 + rs)
    yn = jnp.sqrt(rs + EPS)
    th = _arccosh(jnp.maximum(x0, 1.0 + EPS))
    return th * resr / yn


def _dense1_body(x_ref, w_ref, b_ref, o_ref):
    x = x_ref[...]
    is0 = _col_mask(x.shape)
    u = _logmap0_point(x, is0)
    mu = jnp.dot(u, w_ref[...], preferred_element_type=jnp.float32) + b_ref[...]
    mu = jnp.where(is0, 0.0, mu)
    o_ref[...] = _exp_proj_log(mu)


def _deg_scale(d0, d1):
    isc0 = _col_mask(d0.shape)
    deg = jnp.sum(jnp.where(isc0, d0 + d1, 0.0), axis=1, keepdims=True)
    return 1.0 / jnp.maximum(deg, 1.0)


def _dense2_body(p0_ref, p1_ref, d0_ref, d1_ref, w_ref, b_ref, o_ref):
    agg = p0_ref[...] + p1_ref[...]
    scale = _deg_scale(d0_ref[...], d1_ref[...])
    is0 = _col_mask(agg.shape)
    t = jnp.where(is0, 0.0, agg * scale)
    t = _exp_proj_log(t)
    t = jnp.where(is0, 0.0, jnp.maximum(t, 0.0))
    t = _exp_proj_log(t)
    mu = jnp.dot(t, w_ref[...], preferred_element_type=jnp.float32) + b_ref[...]
    mu = jnp.where(is0, 0.0, mu)
    o_ref[...] = _exp_proj_log(mu)


def _dense3_body(p0_ref, p1_ref, d0_ref, d1_ref, o_ref):
    agg = p0_ref[...] + p1_ref[...]
    scale = _deg_scale(d0_ref[...], d1_ref[...])
    is0 = _col_mask(agg.shape)
    t = jnp.where(is0, 0.0, agg * scale)
    t = _exp_proj_log(t)
    m = jnp.max(t, axis=1, keepdims=True)
    lse = jnp.log(jnp.sum(jnp.exp(t - m), axis=1, keepdims=True))
    o_ref[...] = t - m - lse


def _row_spec(rb):
    return pl.BlockSpec((rb, 128), lambda i: (i, 0))


def _fixed_spec(shape):
    return pl.BlockSpec(shape, lambda i: (0, 0))


def _dense1(x, w1t, b1, rb):
    n = x.shape[0]
    return pl.pallas_call(
        _dense1_body,
        grid=(n // rb,),
        in_specs=[_row_spec(rb), _fixed_spec((128, 128)), _fixed_spec((1, 128))],
        out_specs=_row_spec(rb),
        out_shape=jax.ShapeDtypeStruct((n, 128), jnp.float32),
    )(x, w1t, b1)


def _dense2(p0, p1, d0, d1, w2t, b2, rb, n):
    dspec = pl.BlockSpec((rb, 16), lambda i: (i, 0))
    return pl.pallas_call(
        _dense2_body,
        grid=(n // rb,),
        in_specs=[_row_spec(rb), _row_spec(rb), dspec, dspec,
                  _fixed_spec((128, 128)), _fixed_spec((1, 128))],
        out_specs=_row_spec(rb),
        out_shape=jax.ShapeDtypeStruct((n, 128), jnp.float32),
    )(p0, p1, d0, d1, w2t, b2)


def _dense3(p0, p1, d0, d1, rb, n):
    dspec = pl.BlockSpec((rb, 16), lambda i: (i, 0))
    return pl.pallas_call(
        _dense3_body,
        grid=(n // rb,),
        in_specs=[_row_spec(rb), _row_spec(rb), dspec, dspec],
        out_specs=_row_spec(rb),
        out_shape=jax.ShapeDtypeStruct((n, 128), jnp.float32),
    )(p0, p1, d0, d1)


def _npad(n):
    return ((n + NS * 8 - 1) // (NS * 8)) * NS * 8


@functools.partial(jax.jit, static_argnames=("n", "d", "e"))
def _segsum(src2, dst2, ht, *, n, d, e):
    nch = e // NW // B_EDGE
    epw = nch * B_EDGE
    npad = _npad(n)
    rpt = npad // NS
    mesh = plsc.VectorSubcoreMesh(core_axis_name="c", subcore_axis_name="s")

    out_type = [jax.ShapeDtypeStruct((NC, npad, d), jnp.float32)]
    scratch = [
        pltpu.VMEM((epw,), jnp.int32),
        pltpu.VMEM((epw,), jnp.int32),
        pltpu.VMEM((B_EDGE,), jnp.int32),
        pltpu.VMEM((B_EDGE,), jnp.int32),
        pltpu.VMEM((B_EDGE, d), jnp.float32),
        pltpu.VMEM((B_EDGE, d), jnp.float32),
        pltpu.VMEM((64, d), jnp.float32),
        pltpu.VMEM_SHARED((npad, d), jnp.float32),
        pltpu.SemaphoreType.DMA,
        pltpu.SemaphoreType.DMA,
    ]

    def body(src_hbm, dst_hbm, ht_hbm, out_hbm, sidx, didx, gidx0, gidx1,
             rows0, rows1, zrow, acc, sem0, sem1):
        c = lax.axis_index("c")
        s = lax.axis_index("s")
        wid = c * NS + s

        zero16 = jnp.zeros((16,), jnp.float32)

        def zfill(i, carry):
            for j in range(d // 16):
                zrow[i, pl.ds(j * 16, 16)] = zero16
            return carry
        lax.fori_loop(0, 64, zfill, 0)

        base = s * rpt
        nfull, tail = rpt // 64, rpt % 64
        for k in range(nfull):
            pltpu.sync_copy(zrow, acc.at[pl.ds(base + k * 64, 64)])
        if tail:
            pltpu.sync_copy(zrow.at[pl.ds(0, tail)],
                            acc.at[pl.ds(base + nfull * 64, tail)])
        plsc.subcore_barrier()

        pltpu.sync_copy(src_hbm.at[wid], sidx)
        pltpu.sync_copy(dst_hbm.at[wid], didx)

        def fire(ck, rows, sem):
            pltpu.async_copy(ht_hbm.at[sidx.at[pl.ds(ck * B_EDGE, B_EDGE)]],
                             rows, sem)

        def wait(rows, sem):
            pltpu.make_async_copy(ht_hbm.at[pl.ds(0, B_EDGE)], rows,
                                  sem).wait()

        def scat(ck, rows, gidx):
            eb = ck * B_EDGE
            for j in range(B_EDGE // 16):
                gidx[pl.ds(j * 16, 16)] = didx[pl.ds(eb + j * 16, 16)]
            pltpu.sync_copy(rows, acc.at[gidx], add=True)

        fire(0, rows0, sem0)

        def eloop(k, carry):
            c0 = 2 * k
            fire(c0 + 1, rows1, sem1)
            wait(rows0, sem0)
            scat(c0, rows0, gidx0)
            fire(c0 + 2, rows0, sem0)
            wait(rows1, sem1)
            scat(c0 + 1, rows1, gidx1)
            return carry
        lax.fori_loop(0, (nch - 1) // 2, eloop, 0)
        wait(rows0, sem0)
        scat(nch - 1, rows0, gidx0)
        plsc.subcore_barrier()

        pltpu.sync_copy(acc.at[pl.ds(base, rpt)],
                        out_hbm.at[c, pl.ds(base, rpt)])

    run = pl.kernel(body, out_type=out_type, mesh=mesh, scratch_types=scratch)
    return run(src2, dst2, ht)


@functools.partial(jax.jit, static_argnames=("n", "e"))
def _degree(dst3, *, n, e):
    nch = e // NW // B_EDGE
    npad = _npad(n)
    rpt = npad // NS
    zr = 128
    mesh = plsc.VectorSubcoreMesh(core_axis_name="c", subcore_axis_name="s")

    out_type = [jax.ShapeDtypeStruct((NC, npad, 16), jnp.float32)]
    scratch = [
        pltpu.VMEM((nch, B_EDGE), jnp.int32),
        pltpu.VMEM((B_EDGE, 16), jnp.float32),
        pltpu.VMEM((zr, 16), jnp.float32),
        pltpu.VMEM_SHARED((npad, 16), jnp.float32),
    ]

    def body(dst_hbm, deg_hbm, didx, ones, zdeg, dacc):
        c = lax.axis_index("c")
        s = lax.axis_index("s")
        wid = s * NC + c

        zero16 = jnp.zeros((16,), jnp.float32)
        one16 = jnp.ones((16,), jnp.float32)

        def zfill(i, carry):
            zdeg[i, pl.ds(0, 16)] = zero16
            return carry
        lax.fori_loop(0, zr, zfill, 0)

        def ofill(i, carry):
            ones[i, pl.ds(0, 16)] = one16
            return carry
        lax.fori_loop(0, B_EDGE, ofill, 0)

        base = s * rpt
        for k in range(rpt // zr):
            pltpu.sync_copy(zdeg, dacc.at[pl.ds(base + k * zr, zr)])
        plsc.subcore_barrier()

        pltpu.sync_copy(dst_hbm.at[wid], didx)

        def eloop(i, carry):
            pltpu.sync_copy(ones, dacc.at[didx.at[i]], add=True)
            return carry
        lax.fori_loop(0, nch, eloop, 0)
        plsc.subcore_barrier()

        pltpu.sync_copy(dacc.at[pl.ds(base, rpt)],
                        deg_hbm.at[c, pl.ds(base, rpt)])

    run = pl.kernel(body, out_type=out_type, mesh=mesh, scratch_types=scratch)
    return run(dst3)


def kernel(x, edge_index, W1, b1, W2, b2):
    n, d = x.shape
    e = edge_index.shape[1]
    rb = 1000

    src2 = edge_index[0].reshape(NW, e // NW)
    dst2 = edge_index[1].reshape(NW, e // NW)
    dst3d = edge_index[1].reshape(NW, e // NW // B_EDGE, B_EDGE)

    ht1 = _dense1(x, W1.T, b1.reshape(1, -1), rb)
    (degp,) = _degree(dst3d, n=n, e=e)
    (p1,) = _segsum(src2, dst2, ht1, n=n, d=d, e=e)
    ht2 = _dense2(p1[0], p1[1], degp[0], degp[1], W2.T, b2.reshape(1, -1),
                  rb, n)
    (p2,) = _segsum(src2, dst2, ht2, n=n, d=d, e=e)
    return _dense3(p2[0], p2[1], degp[0], degp[1], rb, n)


# speedup vs baseline: 11.4087x; 1.0815x over previous
import functools

import jax
import jax.numpy as jnp
from jax import lax
from jax.experimental import pallas as pl
from jax.experimental.pallas import tpu as pltpu
from jax.experimental.pallas import tpu_sc as plsc

C = 1.0
EPS = 1e-6

NC = 2
NS = 16
NW = NC * NS
B_EDGE = 80


def _col_mask(shape):
    return lax.broadcasted_iota(jnp.int32, shape, 1) == 0


def _arccosh(z):
    return jnp.log(z + jnp.sqrt(z * z - 1.0))


def _logmap0_point(x, is0):
    y = jnp.where(is0, 0.0, x)
    x0 = jnp.sum(jnp.where(is0, x, 0.0), axis=1, keepdims=True)
    yn = jnp.sqrt(jnp.sum(y * y, axis=1, keepdims=True) + EPS)
    th = _arccosh(jnp.maximum(x0, 1.0 + EPS))
    return th * y / yn


def _dense1_body(x_ref, w_ref, b_ref, o_ref):
    x = x_ref[...]
    is0 = _col_mask(x.shape)
    u = _logmap0_point(x, is0)
    mu = jnp.dot(u, w_ref[...], preferred_element_type=jnp.float32) + b_ref[...]
    o_ref[...] = jnp.where(is0, 0.0, mu)


def _deg_scale(d0, d1):
    isc0 = _col_mask(d0.shape)
    deg = jnp.sum(jnp.where(isc0, d0 + d1, 0.0), axis=1, keepdims=True)
    return 1.0 / jnp.maximum(deg, 1.0)


def _dense2_body(p0_ref, p1_ref, d0_ref, d1_ref, w_ref, b_ref, o_ref):
    agg = p0_ref[...] + p1_ref[...]
    scale = _deg_scale(d0_ref[...], d1_ref[...])
    is0 = _col_mask(agg.shape)
    t = jnp.where(is0, 0.0, agg * scale)
    t = jnp.maximum(t, 0.0)
    mu = jnp.dot(t, w_ref[...], preferred_element_type=jnp.float32) + b_ref[...]
    o_ref[...] = jnp.where(is0, 0.0, mu)


def _dense3_body(p0_ref, p1_ref, d0_ref, d1_ref, o_ref):
    agg = p0_ref[...] + p1_ref[...]
    scale = _deg_scale(d0_ref[...], d1_ref[...])
    is0 = _col_mask(agg.shape)
    t = jnp.where(is0, 0.0, agg * scale)
    m = jnp.max(t, axis=1, keepdims=True)
    lse = jnp.log(jnp.sum(jnp.exp(t - m), axis=1, keepdims=True))
    o_ref[...] = t - m - lse


def _row_spec(rb):
    return pl.BlockSpec((rb, 128), lambda i: (i, 0))


def _fixed_spec(shape):
    return pl.BlockSpec(shape, lambda i: (0, 0))


def _dense1(x, w1t, b1, rb):
    n = x.shape[0]
    return pl.pallas_call(
        _dense1_body,
        grid=(n // rb,),
        in_specs=[_row_spec(rb), _fixed_spec((128, 128)), _fixed_spec((1, 128))],
        out_specs=_row_spec(rb),
        out_shape=jax.ShapeDtypeStruct((n, 128), jnp.float32),
    )(x, w1t, b1)


def _dense2(p0, p1, d0, d1, w2t, b2, rb, n):
    dspec = pl.BlockSpec((rb, 16), lambda i: (i, 0))
    return pl.pallas_call(
        _dense2_body,
        grid=(n // rb,),
        in_specs=[_row_spec(rb), _row_spec(rb), dspec, dspec,
                  _fixed_spec((128, 128)), _fixed_spec((1, 128))],
        out_specs=_row_spec(rb),
        out_shape=jax.ShapeDtypeStruct((n, 128), jnp.float32),
    )(p0, p1, d0, d1, w2t, b2)


def _dense3(p0, p1, d0, d1, rb, n):
    dspec = pl.BlockSpec((rb, 16), lambda i: (i, 0))
    return pl.pallas_call(
        _dense3_body,
        grid=(n // rb,),
        in_specs=[_row_spec(rb), _row_spec(rb), dspec, dspec],
        out_specs=_row_spec(rb),
        out_shape=jax.ShapeDtypeStruct((n, 128), jnp.float32),
    )(p0, p1, d0, d1)


def _npad(n):
    return ((n + NS * 8 - 1) // (NS * 8)) * NS * 8


@functools.partial(jax.jit, static_argnames=("n", "d", "e"))
def _segsum(src2, dst2, ht, *, n, d, e):
    nch = e // NW // B_EDGE
    epw = nch * B_EDGE
    npad = _npad(n)
    rpt = npad // NS
    mesh = plsc.VectorSubcoreMesh(core_axis_name="c", subcore_axis_name="s")

    out_type = [jax.ShapeDtypeStruct((NC, npad, d), jnp.float32)]
    scratch = [
        pltpu.VMEM((epw,), jnp.int32),
        pltpu.VMEM((epw,), jnp.int32),
        pltpu.VMEM((B_EDGE,), jnp.int32),
        pltpu.VMEM((B_EDGE,), jnp.int32),
        pltpu.VMEM((B_EDGE, d), jnp.float32),
        pltpu.VMEM((B_EDGE, d), jnp.float32),
        pltpu.VMEM((64, d), jnp.float32),
        pltpu.VMEM_SHARED((npad, d), jnp.float32),
        pltpu.SemaphoreType.DMA,
        pltpu.SemaphoreType.DMA,
    ]

    def body(src_hbm, dst_hbm, ht_hbm, out_hbm, sidx, didx, gidx0, gidx1,
             rows0, rows1, zrow, acc, sem0, sem1):
        c = lax.axis_index("c")
        s = lax.axis_index("s")
        wid = c * NS + s

        zero16 = jnp.zeros((16,), jnp.float32)

        def zfill(i, carry):
            for j in range(d // 16):
                zrow[i, pl.ds(j * 16, 16)] = zero16
            return carry
        lax.fori_loop(0, 64, zfill, 0)

        base = s * rpt
        nfull, tail = rpt // 64, rpt % 64
        for k in range(nfull):
            pltpu.sync_copy(zrow, acc.at[pl.ds(base + k * 64, 64)])
        if tail:
            pltpu.sync_copy(zrow.at[pl.ds(0, tail)],
                            acc.at[pl.ds(base + nfull * 64, tail)])
        plsc.subcore_barrier()

        pltpu.sync_copy(src_hbm.at[wid], sidx)
        pltpu.sync_copy(dst_hbm.at[wid], didx)

        def fire(ck, rows, sem):
            pltpu.async_copy(ht_hbm.at[sidx.at[pl.ds(ck * B_EDGE, B_EDGE)]],
                             rows, sem)

        def wait(rows, sem):
            pltpu.make_async_copy(ht_hbm.at[pl.ds(0, B_EDGE)], rows,
                                  sem).wait()

        def scat(ck, rows, gidx):
            eb = ck * B_EDGE
            for j in range(B_EDGE // 16):
                gidx[pl.ds(j * 16, 16)] = didx[pl.ds(eb + j * 16, 16)]
            pltpu.sync_copy(rows, acc.at[gidx], add=True)

        fire(0, rows0, sem0)

        def eloop(k, carry):
            c0 = 2 * k
            fire(c0 + 1, rows1, sem1)
            wait(rows0, sem0)
            scat(c0, rows0, gidx0)
            fire(c0 + 2, rows0, sem0)
            wait(rows1, sem1)
            scat(c0 + 1, rows1, gidx1)
            return carry
        lax.fori_loop(0, (nch - 1) // 2, eloop, 0)
        wait(rows0, sem0)
        scat(nch - 1, rows0, gidx0)
        plsc.subcore_barrier()

        pltpu.sync_copy(acc.at[pl.ds(base, rpt)],
                        out_hbm.at[c, pl.ds(base, rpt)])

    run = pl.kernel(body, out_type=out_type, mesh=mesh, scratch_types=scratch)
    return run(src2, dst2, ht)


@functools.partial(jax.jit, static_argnames=("n", "e"))
def _degree(dst3, *, n, e):
    nch = e // NW // B_EDGE
    npad = _npad(n)
    rpt = npad // NS
    zr = 128
    mesh = plsc.VectorSubcoreMesh(core_axis_name="c", subcore_axis_name="s")

    out_type = [jax.ShapeDtypeStruct((NC, npad, 16), jnp.float32)]
    scratch = [
        pltpu.VMEM((nch, B_EDGE), jnp.int32),
        pltpu.VMEM((B_EDGE, 16), jnp.float32),
        pltpu.VMEM((zr, 16), jnp.float32),
        pltpu.VMEM_SHARED((npad, 16), jnp.float32),
    ]

    def body(dst_hbm, deg_hbm, didx, ones, zdeg, dacc):
        c = lax.axis_index("c")
        s = lax.axis_index("s")
        wid = s * NC + c

        zero16 = jnp.zeros((16,), jnp.float32)
        one16 = jnp.ones((16,), jnp.float32)

        def zfill(i, carry):
            zdeg[i, pl.ds(0, 16)] = zero16
            return carry
        lax.fori_loop(0, zr, zfill, 0)

        def ofill(i, carry):
            ones[i, pl.ds(0, 16)] = one16
            return carry
        lax.fori_loop(0, B_EDGE, ofill, 0)

        base = s * rpt
        for k in range(rpt // zr):
            pltpu.sync_copy(zdeg, dacc.at[pl.ds(base + k * zr, zr)])
        plsc.subcore_barrier()

        pltpu.sync_copy(dst_hbm.at[wid], didx)

        def eloop(i, carry):
            pltpu.sync_copy(ones, dacc.at[didx.at[i]], add=True)
            return carry
        lax.fori_loop(0, nch, eloop, 0)
        plsc.subcore_barrier()

        pltpu.sync_copy(dacc.at[pl.ds(base, rpt)],
                        deg_hbm.at[c, pl.ds(base, rpt)])

    run = pl.kernel(body, out_type=out_type, mesh=mesh, scratch_types=scratch)
    return run(dst3)


def kernel(x, edge_index, W1, b1, W2, b2):
    n, d = x.shape
    e = edge_index.shape[1]
    rb = 1000

    src2 = edge_index[0].reshape(NW, e // NW)
    dst2 = edge_index[1].reshape(NW, e // NW)
    dst3d = edge_index[1].reshape(NW, e // NW // B_EDGE, B_EDGE)

    ht1 = _dense1(x, W1.T, b1.reshape(1, -1), rb)
    (degp,) = _degree(dst3d, n=n, e=e)
    (p1,) = _segsum(src2, dst2, ht1, n=n, d=d, e=e)
    ht2 = _dense2(p1[0], p1[1], degp[0], degp[1], W2.T, b2.reshape(1, -1),
                  rb, n)
    (p2,) = _segsum(src2, dst2, ht2, n=n, d=d, e=e)
    return _dense3(p2[0], p2[1], degp[0], degp[1], rb, n)
